# baseline (speedup 1.0000x reference)
import functools

import jax
import jax.numpy as jnp
import numpy as np
from jax import lax
from jax.experimental import pallas as pl
from jax.experimental.pallas import tpu as pltpu
from jax.experimental.pallas import tpu_sc as plsc

_N = 10000
_E = 160000
_EP = 163840
_NP = 10240
_L = 16
_CH = 128
_NW = 32
_PW = _EP // _NW
_NCH = _PW // _CH
_ROWS_PER_SUB = _NP // 16

@functools.cache
def _get_sc_gather():
    mesh = plsc.VectorSubcoreMesh(core_axis_name="c", subcore_axis_name="s")

    @functools.partial(
        pl.kernel,
        out_type=jax.ShapeDtypeStruct((_EP, _L), jnp.float32),
        mesh=mesh,
        scratch_types=[
            pltpu.VMEM((_NCH, _CH), jnp.int32),
            pltpu.VMEM((_PW, _L), jnp.float32),
            pltpu.SemaphoreType.DMA,
        ],
        compiler_params=pltpu.CompilerParams(use_tc_tiling_on_sc=False),
    )
    def _sc_gather(x_hbm, idx_hbm, out_hbm, idx_v, rows_v, sem):
        wid = lax.axis_index("s") * 2 + lax.axis_index("c")
        pltpu.sync_copy(idx_hbm.at[pl.ds(wid * _NCH, _NCH)], idx_v)

        @pl.loop(0, _NCH)
        def _fire(j):
            pltpu.async_copy(x_hbm.at[idx_v.at[j]], rows_v.at[pl.ds(j * _CH, _CH)], sem)

        @pl.loop(0, _NCH)
        def _drain(j):
            pltpu.make_async_copy(
                x_hbm.at[idx_v.at[j]], rows_v.at[pl.ds(j * _CH, _CH)], sem
            ).wait()

        pltpu.sync_copy(rows_v, out_hbm.at[pl.ds(wid * _PW, _PW)])

    return _sc_gather


@functools.cache
def _make_sc_scatter(with_deg):
    mesh = plsc.VectorSubcoreMesh(core_axis_name="c", subcore_axis_name="s")
    n_out = 2 if with_deg else 1
    scratch = [
        pltpu.VMEM((_NCH, _CH), jnp.int32),
        pltpu.VMEM((_PW, _L), jnp.float32),
        pltpu.VMEM_SHARED((_NP, _L), jnp.float32),
    ]
    if with_deg:
        scratch.append(pltpu.VMEM_SHARED((_NP, _L), jnp.float32))
        scratch.append(pltpu.VMEM((_CH, _L), jnp.float32))

    out_sds = jax.ShapeDtypeStruct((2, _NP, _L), jnp.float32)

    @functools.partial(
        pl.kernel,
        out_type=(out_sds,) * n_out if with_deg else out_sds,
        mesh=mesh,
        scratch_types=scratch,
        compiler_params=pltpu.CompilerParams(use_tc_tiling_on_sc=False),
    )
    def _sc_scatter(msg_hbm, idx_hbm, z_hbm, ones_hbm, *rest):
        if with_deg:
            agg_out, deg_out, idx_v, rows_v, agg_sh, deg_sh, ones_v = rest
        else:
            agg_out, idx_v, rows_v, agg_sh = rest
        c = lax.axis_index("c")
        s = lax.axis_index("s")
        wid = s * 2 + c
        zslc = pl.ds(s * _ROWS_PER_SUB, _ROWS_PER_SUB)
        pltpu.sync_copy(z_hbm.at[zslc], agg_sh.at[zslc])
        if with_deg:
            pltpu.sync_copy(z_hbm.at[zslc], deg_sh.at[zslc])
            pltpu.sync_copy(ones_hbm, ones_v)
        plsc.subcore_barrier()

        pltpu.sync_copy(idx_hbm.at[pl.ds(wid * _NCH, _NCH)], idx_v)
        pltpu.sync_copy(msg_hbm.at[pl.ds(wid * _PW, _PW)], rows_v)

        @pl.loop(0, _NCH)
        def _scat(j):
            pltpu.sync_copy(
                rows_v.at[pl.ds(j * _CH, _CH)], agg_sh.at[idx_v.at[j]], add=True
            )
            if with_deg:
                pltpu.sync_copy(ones_v, deg_sh.at[idx_v.at[j]], add=True)

        plsc.subcore_barrier()
        pltpu.sync_copy(agg_sh.at[zslc], agg_out.at[c, zslc])
        if with_deg:
            pltpu.sync_copy(deg_sh.at[zslc], deg_out.at[c, zslc])

    return _sc_scatter


_ET = 4096


def _tc_edge_body(ea_ref, xj_ref, w1, b1, w2, b2, w3, b3, r_ref, s_ref, out_ref):
    f32 = jnp.float32
    bf = jnp.bfloat16
    h = jax.nn.gelu(jnp.dot(ea_ref[...].astype(bf), w1[...].astype(bf),
                            preferred_element_type=f32) + b1[...])
    h = jax.nn.gelu(jnp.dot(h.astype(bf), w2[...].astype(bf),
                            preferred_element_type=f32) + b2[...])
    w = jnp.dot(h.astype(bf), w3[...].astype(bf), preferred_element_type=f32) + b3[...]
    xr = jnp.dot(xj_ref[...].astype(bf), r_ref[...].astype(bf),
                 preferred_element_type=f32)
    out_ref[...] = jnp.dot((xr * w).astype(bf), s_ref[...].astype(bf),
                           preferred_element_type=f32)


def _tc_edge(ea, xj, kW1, b1, kW2, b2, kW3, b3, Rm, Sm):
    grid = (_EP // _ET,)
    c0 = lambda i: (0, 0)
    return pl.pallas_call(
        _tc_edge_body,
        grid=grid,
        in_specs=[
            pl.BlockSpec((_ET, _L), lambda i: (i, 0)),
            pl.BlockSpec((_ET, _L), lambda i: (i, 0)),
            pl.BlockSpec((16, 64), c0),
            pl.BlockSpec((1, 64), c0),
            pl.BlockSpec((64, 64), c0),
            pl.BlockSpec((1, 64), c0),
            pl.BlockSpec((64, 256), c0),
            pl.BlockSpec((1, 256), c0),
            pl.BlockSpec((16, 256), c0),
            pl.BlockSpec((256, 16), c0),
        ],
        out_specs=pl.BlockSpec((_ET, _L), lambda i: (i, 0)),
        out_shape=jax.ShapeDtypeStruct((_EP, _L), jnp.float32),
    )(ea, xj, kW1, b1, kW2, b2, kW3, b3, Rm, Sm)


def _tc_root_body(x_ref, a_ref, b_ref, da_ref, db_ref, r_ref, bias_ref, out_ref):
    deg = jnp.maximum(da_ref[...] + db_ref[...], 1.0)
    agg = (a_ref[...] + b_ref[...]) / deg
    xw = jnp.dot(x_ref[...], r_ref[...], preferred_element_type=jnp.float32)
    out_ref[...] = jax.nn.gelu(xw + agg + bias_ref[...])


def _tc_root(x, agg_a, agg_b, deg_a, deg_b, root, bias):
    full = lambda i: (0, 0)
    return pl.pallas_call(
        _tc_root_body,
        grid=(1,),
        in_specs=[
            pl.BlockSpec((_N, _L), full),
            pl.BlockSpec((_N, _L), full),
            pl.BlockSpec((_N, _L), full),
            pl.BlockSpec((_N, _L), full),
            pl.BlockSpec((_N, _L), full),
            pl.BlockSpec((_L, _L), full),
            pl.BlockSpec((1, _L), full),
        ],
        out_specs=pl.BlockSpec((_N, _L), full),
        out_shape=jax.ShapeDtypeStruct((_N, _L), jnp.float32),
    )(x, agg_a, agg_b, deg_a, deg_b, root, bias)


def kernel(nodes, edge_index, edge_attr, kW1, kb1, kW2, kb2, kW3, kb3,
           root0, bias0, root1, bias1):
    pad = _EP - _E
    src = edge_index[0].astype(jnp.int32)
    dst = edge_index[1].astype(jnp.int32)
    src_p = jnp.concatenate([src, jnp.zeros((pad,), jnp.int32)]).reshape(_EP // _CH, _CH)
    dst_p = jnp.concatenate([dst, jnp.full((pad,), _N, jnp.int32)]).reshape(_EP // _CH, _CH)
    ea_p = jnp.concatenate([edge_attr, jnp.zeros((pad, _L), jnp.float32)])

    Rm = jnp.asarray((np.arange(256)[None, :] // 16 == np.arange(16)[:, None]).astype(np.float32))
    Sm = jnp.asarray((np.arange(256)[:, None] % 16 == np.arange(16)[None, :]).astype(np.float32))
    z = jnp.zeros((_NP, _L), jnp.float32)
    ones = jnp.ones((_CH, _L), jnp.float32)
    b1 = kb1.reshape(1, 64)
    b2 = kb2.reshape(1, 64)
    b3 = kb3.reshape(1, 256)

    x = nodes
    degp = None
    for root, bias, first in ((root0, bias0, True), (root1, bias1, False)):
        xj = _get_sc_gather()(x, src_p)
        msg = _tc_edge(ea_p, xj, kW1, b1, kW2, b2, kW3, b3, Rm, Sm)
        if first:
            aggp, degp = _make_sc_scatter(True)(msg, dst_p, z, ones)
        else:
            aggp = _make_sc_scatter(False)(msg, dst_p, z, ones)
        x = _tc_root(x, aggp[0, :_N], aggp[1, :_N], degp[0, :_N], degp[1, :_N],
                     root, bias.reshape(1, _L))
    return x


# speedup vs baseline: 1.7176x; 1.7176x over previous
import functools

import jax
import jax.numpy as jnp
import numpy as np
from jax import lax
from jax.experimental import pallas as pl
from jax.experimental.pallas import tpu as pltpu
from jax.experimental.pallas import tpu_sc as plsc

_N = 10000
_E = 160000
_NP = 10240
_L = 16
_CH = 125
_NW = 32
_PW = _E // _NW
_NCH = _PW // _CH
_ROWS_PER_SUB = _NP // 16


@functools.cache
def _get_sc_gather():
    mesh = plsc.VectorSubcoreMesh(core_axis_name="c", subcore_axis_name="s")

    @functools.partial(
        pl.kernel,
        out_type=jax.ShapeDtypeStruct((_E, _L), jnp.float32),
        mesh=mesh,
        scratch_types=[
            pltpu.VMEM((_NCH, _CH), jnp.int32),
            pltpu.VMEM((_PW, _L), jnp.float32),
            pltpu.SemaphoreType.DMA,
        ],
        compiler_params=pltpu.CompilerParams(use_tc_tiling_on_sc=False),
    )
    def _sc_gather(x_hbm, idx_hbm, out_hbm, idx_v, rows_v, sem):
        wid = lax.axis_index("s") * 2 + lax.axis_index("c")
        pltpu.sync_copy(idx_hbm.at[pl.ds(wid * _NCH, _NCH)], idx_v)

        @pl.loop(0, _NCH)
        def _fire(j):
            pltpu.async_copy(x_hbm.at[idx_v.at[j]], rows_v.at[pl.ds(j * _CH, _CH)], sem)

        @pl.loop(0, _NCH)
        def _drain(j):
            pltpu.make_async_copy(
                x_hbm.at[idx_v.at[j]], rows_v.at[pl.ds(j * _CH, _CH)], sem
            ).wait()

        pltpu.sync_copy(rows_v, out_hbm.at[pl.ds(wid * _PW, _PW)])

    return _sc_gather


@functools.cache
def _make_sc_scatter(with_deg):
    mesh = plsc.VectorSubcoreMesh(core_axis_name="c", subcore_axis_name="s")
    scratch = [
        pltpu.VMEM((_NCH, _CH), jnp.int32),
        pltpu.VMEM((_PW, _L), jnp.float32),
        pltpu.VMEM_SHARED((_NP, _L), jnp.float32),
    ]
    if with_deg:
        scratch.append(pltpu.VMEM_SHARED((_NP, _L), jnp.float32))
        scratch.append(pltpu.VMEM((_CH, _L), jnp.float32))

    out_sds = jax.ShapeDtypeStruct((2, _NP, _L), jnp.float32)

    @functools.partial(
        pl.kernel,
        out_type=(out_sds, out_sds) if with_deg else out_sds,
        mesh=mesh,
        scratch_types=scratch,
        compiler_params=pltpu.CompilerParams(use_tc_tiling_on_sc=False),
    )
    def _sc_scatter(msg_hbm, idx_hbm, z_hbm, ones_hbm, *rest):
        if with_deg:
            agg_out, deg_out, idx_v, rows_v, agg_sh, deg_sh, ones_v = rest
        else:
            agg_out, idx_v, rows_v, agg_sh = rest
        c = lax.axis_index("c")
        s = lax.axis_index("s")
        wid = s * 2 + c
        zslc = pl.ds(s * _ROWS_PER_SUB, _ROWS_PER_SUB)
        pltpu.sync_copy(z_hbm.at[zslc], agg_sh.at[zslc])
        if with_deg:
            pltpu.sync_copy(z_hbm.at[zslc], deg_sh.at[zslc])
            pltpu.sync_copy(ones_hbm, ones_v)
        plsc.subcore_barrier()

        pltpu.sync_copy(idx_hbm.at[pl.ds(wid * _NCH, _NCH)], idx_v)
        pltpu.sync_copy(msg_hbm.at[pl.ds(wid * _PW, _PW)], rows_v)

        @pl.loop(0, _NCH)
        def _scat(j):
            pltpu.sync_copy(
                rows_v.at[pl.ds(j * _CH, _CH)], agg_sh.at[idx_v.at[j]], add=True
            )
            if with_deg:
                pltpu.sync_copy(ones_v, deg_sh.at[idx_v.at[j]], add=True)

        plsc.subcore_barrier()
        pltpu.sync_copy(agg_sh.at[zslc], agg_out.at[c, zslc])
        if with_deg:
            pltpu.sync_copy(deg_sh.at[zslc], deg_out.at[c, zslc])

    return _sc_scatter


_RB = 800
_EB = _RB * 8
_GRID = (_E * _L) // (128 * _RB)


def _unpack(v):
    return jnp.concatenate([v[:, 16 * j:16 * (j + 1)] for j in range(8)], axis=0)


def _pack(v):
    return jnp.concatenate([v[_RB * j:_RB * (j + 1), :] for j in range(8)], axis=1)


def _tc_edge_body(ea_ref, xj_ref, w1, b1, w2, b2, w3, b3, r_ref, s_ref, out_ref):
    f32 = jnp.float32
    bf = jnp.bfloat16
    ea = _unpack(ea_ref[...])
    xj = _unpack(xj_ref[...])
    h = jax.nn.gelu(jnp.dot(ea.astype(bf), w1[...], preferred_element_type=f32)
                    + b1[...])
    h = jax.nn.gelu(jnp.dot(h.astype(bf), w2[...], preferred_element_type=f32)
                    + b2[...])
    w = jnp.dot(h.astype(bf), w3[...], preferred_element_type=f32) + b3[...]
    xr = jnp.dot(xj.astype(bf), r_ref[...], preferred_element_type=f32)
    msg = jnp.dot((xr * w).astype(bf), s_ref[...], preferred_element_type=f32)
    out_ref[...] = _pack(msg)


def _tc_edge(ea, xj, kW1, b1, kW2, b2, kW3, b3, Rm, Sm):
    c0 = lambda i: (0, 0)
    return pl.pallas_call(
        _tc_edge_body,
        grid=(_GRID,),
        in_specs=[
            pl.BlockSpec((_RB, 128), lambda i: (i, 0)),
            pl.BlockSpec((_RB, 128), lambda i: (i, 0)),
            pl.BlockSpec((16, 64), c0),
            pl.BlockSpec((1, 64), c0),
            pl.BlockSpec((64, 64), c0),
            pl.BlockSpec((1, 64), c0),
            pl.BlockSpec((64, 256), c0),
            pl.BlockSpec((1, 256), c0),
            pl.BlockSpec((16, 256), c0),
            pl.BlockSpec((256, 16), c0),
        ],
        out_specs=pl.BlockSpec((_RB, 128), lambda i: (i, 0)),
        out_shape=jax.ShapeDtypeStruct(((_E * _L) // 128, 128), jnp.float32),
    )(ea, xj, kW1, b1, kW2, b2, kW3, b3, Rm, Sm)


_NR = (_N * _L) // 128


def _tc_root_body(x_ref, a_ref, b_ref, da_ref, db_ref, rk_ref, bias_ref, out_ref):
    deg = jnp.maximum(da_ref[...] + db_ref[...], 1.0)
    agg = (a_ref[...] + b_ref[...]) / deg
    xw = jnp.dot(x_ref[...], rk_ref[...], preferred_element_type=jnp.float32)
    out_ref[...] = jax.nn.gelu(xw + agg + bias_ref[...])


def _tc_root(x, agg_a, agg_b, deg_a, deg_b, rootk, bias128):
    full = lambda: (0, 0)
    return pl.pallas_call(
        _tc_root_body,
        grid=(),
        in_specs=[
            pl.BlockSpec((_NR, 128), full),
            pl.BlockSpec((_NR, 128), full),
            pl.BlockSpec((_NR, 128), full),
            pl.BlockSpec((_NR, 128), full),
            pl.BlockSpec((_NR, 128), full),
            pl.BlockSpec((128, 128), full),
            pl.BlockSpec((1, 128), full),
        ],
        out_specs=pl.BlockSpec((_NR, 128), full),
        out_shape=jax.ShapeDtypeStruct((_NR, 128), jnp.float32),
    )(x, agg_a, agg_b, deg_a, deg_b, rootk, bias128)


def kernel(nodes, edge_index, edge_attr, kW1, kb1, kW2, kb2, kW3, kb3,
           root0, bias0, root1, bias1):
    src2d = edge_index[0].astype(jnp.int32).reshape(_E // _CH, _CH)
    dst2d = edge_index[1].astype(jnp.int32).reshape(_E // _CH, _CH)
    ea128 = edge_attr.reshape((_E * _L) // 128, 128)

    bf = jnp.bfloat16
    Rm = jnp.asarray((np.arange(256)[None, :] // 16 == np.arange(16)[:, None])
                     .astype(np.float32), dtype=bf)
    Sm = jnp.asarray((np.arange(256)[:, None] % 16 == np.arange(16)[None, :])
                     .astype(np.float32), dtype=bf)
    z = jnp.zeros((_NP, _L), jnp.float32)
    ones = jnp.ones((_CH, _L), jnp.float32)
    b1 = kb1.reshape(1, 64)
    b2 = kb2.reshape(1, 64)
    b3 = kb3.reshape(1, 256)
    eye8 = jnp.eye(8, dtype=jnp.float32)

    x128 = nodes.reshape(_NR, 128)
    degp = None
    for root, bias, first in ((root0, bias0, True), (root1, bias1, False)):
        xj = _get_sc_gather()(x128.reshape(_N, _L), src2d)
        msg = _tc_edge(ea128, xj.reshape((_E * _L) // 128, 128),
                       kW1.astype(bf), b1, kW2.astype(bf), b2,
                       kW3.astype(bf), b3, Rm, Sm)
        if first:
            aggp, degp = _make_sc_scatter(True)(msg.reshape(_E, _L), dst2d, z, ones)
        else:
            aggp = _make_sc_scatter(False)(msg.reshape(_E, _L), dst2d, z, ones)
        aggp = aggp.reshape(2, (_NP * _L) // 128, 128)
        degr = degp.reshape(2, (_NP * _L) // 128, 128)
        rootk = jnp.kron(eye8, root)
        bias128 = jnp.tile(bias, 8).reshape(1, 128)
        x128 = _tc_root(x128, aggp[0, :_NR], aggp[1, :_NR],
                        degr[0, :_NR], degr[1, :_NR], rootk, bias128)
    return x128.reshape(_N, _L)


# speedup vs baseline: 1.9462x; 1.1331x over previous
import functools

import jax
import jax.numpy as jnp
import numpy as np
from jax import lax
from jax.experimental import pallas as pl
from jax.experimental.pallas import tpu as pltpu
from jax.experimental.pallas import tpu_sc as plsc

_N = 10000
_E = 160000
_NP = 10240
_L = 16
_CH = 125
_NW = 32
_PW = _E // _NW
_NCH = _PW // _CH
_ROWS_PER_SUB = _NP // 16


@functools.cache
def _get_sc_gather():
    mesh = plsc.VectorSubcoreMesh(core_axis_name="c", subcore_axis_name="s")

    @functools.partial(
        pl.kernel,
        out_type=jax.ShapeDtypeStruct((_E, _L), jnp.float32),
        mesh=mesh,
        scratch_types=[
            pltpu.VMEM((_NCH, _CH), jnp.int32),
            pltpu.VMEM((_PW, _L), jnp.float32),
            pltpu.SemaphoreType.DMA,
        ],
        compiler_params=pltpu.CompilerParams(use_tc_tiling_on_sc=False),
    )
    def _sc_gather(x_hbm, idx_hbm, out_hbm, idx_v, rows_v, sem):
        wid = lax.axis_index("s") * 2 + lax.axis_index("c")
        pltpu.sync_copy(idx_hbm.at[pl.ds(wid * _NCH, _NCH)], idx_v)

        @pl.loop(0, _NCH)
        def _fire(j):
            pltpu.async_copy(x_hbm.at[idx_v.at[j]], rows_v.at[pl.ds(j * _CH, _CH)], sem)

        @pl.loop(0, _NCH)
        def _drain(j):
            pltpu.make_async_copy(
                x_hbm.at[idx_v.at[j]], rows_v.at[pl.ds(j * _CH, _CH)], sem
            ).wait()

        pltpu.sync_copy(rows_v, out_hbm.at[pl.ds(wid * _PW, _PW)])

    return _sc_gather


@functools.cache
def _make_sc_scatter(with_deg):
    mesh = plsc.VectorSubcoreMesh(core_axis_name="c", subcore_axis_name="s")
    scratch = [
        pltpu.VMEM((_NCH, _CH), jnp.int32),
        pltpu.VMEM((_PW, _L), jnp.float32),
        pltpu.VMEM_SHARED((_NP, _L), jnp.float32),
    ]
    if with_deg:
        scratch.append(pltpu.VMEM_SHARED((_NP, _L), jnp.float32))
        scratch.append(pltpu.VMEM((_CH, _L), jnp.float32))

    out_sds = jax.ShapeDtypeStruct((2, _NP, _L), jnp.float32)

    @functools.partial(
        pl.kernel,
        out_type=(out_sds, out_sds) if with_deg else out_sds,
        mesh=mesh,
        scratch_types=scratch,
        compiler_params=pltpu.CompilerParams(use_tc_tiling_on_sc=False),
    )
    def _sc_scatter(msg_hbm, idx_hbm, z_hbm, ones_hbm, *rest):
        if with_deg:
            agg_out, deg_out, idx_v, rows_v, agg_sh, deg_sh, ones_v = rest
        else:
            agg_out, idx_v, rows_v, agg_sh = rest
        c = lax.axis_index("c")
        s = lax.axis_index("s")
        wid = s * 2 + c
        zslc = pl.ds(s * _ROWS_PER_SUB, _ROWS_PER_SUB)
        pltpu.sync_copy(z_hbm.at[zslc], agg_sh.at[zslc])
        if with_deg:
            pltpu.sync_copy(z_hbm.at[zslc], deg_sh.at[zslc])
            pltpu.sync_copy(ones_hbm, ones_v)
        plsc.subcore_barrier()

        pltpu.sync_copy(idx_hbm.at[pl.ds(wid * _NCH, _NCH)], idx_v)
        pltpu.sync_copy(msg_hbm.at[pl.ds(wid * _PW, _PW)], rows_v)

        @pl.loop(0, _NCH)
        def _scat(j):
            pltpu.sync_copy(
                rows_v.at[pl.ds(j * _CH, _CH)], agg_sh.at[idx_v.at[j]], add=True
            )
            if with_deg:
                pltpu.sync_copy(ones_v, deg_sh.at[idx_v.at[j]], add=True)

        plsc.subcore_barrier()
        pltpu.sync_copy(agg_sh.at[zslc], agg_out.at[c, zslc])
        if with_deg:
            pltpu.sync_copy(deg_sh.at[zslc], deg_out.at[c, zslc])

    return _sc_scatter


_RB = 800
_EB = _RB * 8
_GRID = (_E * _L) // (128 * _RB)


def _unpack(v, g=16):
    return jnp.concatenate([v[:, g * j:g * (j + 1)] for j in range(8)], axis=0)


def _pack(v):
    return jnp.concatenate([v[_RB * j:_RB * (j + 1), :] for j in range(8)], axis=1)


def _bf(v):
    return v.astype(jnp.bfloat16)


def _matvec(xj128, w, r_ref, s_ref):
    xj = _bf(_unpack(xj128))
    xr = _bf(jnp.dot(xj, r_ref[...], preferred_element_type=jnp.float32))
    return jnp.dot(xr * w, s_ref[...], preferred_element_type=jnp.float32)


def _tc_edge0_body(ea_ref, xj_ref, w1, b1, w2, b2, w3, b3, r_ref, s_ref,
                   out_ref, h2_ref):
    f32 = jnp.float32
    ea = _bf(_unpack(ea_ref[...]))
    h = jax.nn.gelu(_bf(jnp.dot(ea, w1[...], preferred_element_type=f32)) + b1[...])
    h = jax.nn.gelu(_bf(jnp.dot(h, w2[...], preferred_element_type=f32)) + b2[...])
    w = _bf(jnp.dot(h, w3[...], preferred_element_type=f32)) + b3[...]
    out_ref[...] = _pack(_matvec(xj_ref[...], w, r_ref, s_ref))
    h2_ref[...] = _pack(h)


def _tc_edge1_body(h2_ref, xj_ref, w3, b3, r_ref, s_ref, out_ref):
    h = _unpack(h2_ref[...], g=64)
    w = _bf(jnp.dot(h, w3[...], preferred_element_type=jnp.float32)) + b3[...]
    out_ref[...] = _pack(_matvec(xj_ref[...], w, r_ref, s_ref))


def _tc_edge0(ea, xj, kW1, b1, kW2, b2, kW3, b3, Rm, Sm):
    c0 = lambda i: (0, 0)
    return pl.pallas_call(
        _tc_edge0_body,
        grid=(_GRID,),
        in_specs=[
            pl.BlockSpec((_RB, 128), lambda i: (i, 0)),
            pl.BlockSpec((_RB, 128), lambda i: (i, 0)),
            pl.BlockSpec((16, 64), c0),
            pl.BlockSpec((1, 64), c0),
            pl.BlockSpec((64, 64), c0),
            pl.BlockSpec((1, 64), c0),
            pl.BlockSpec((64, 256), c0),
            pl.BlockSpec((1, 256), c0),
            pl.BlockSpec((16, 256), c0),
            pl.BlockSpec((256, 16), c0),
        ],
        out_specs=[
            pl.BlockSpec((_RB, 128), lambda i: (i, 0)),
            pl.BlockSpec((_RB, 512), lambda i: (i, 0)),
        ],
        out_shape=[
            jax.ShapeDtypeStruct(((_E * _L) // 128, 128), jnp.float32),
            jax.ShapeDtypeStruct(((_E * _L) // 128, 512), jnp.bfloat16),
        ],
    )(ea, xj, kW1, b1, kW2, b2, kW3, b3, Rm, Sm)


def _tc_edge1(h2, xj, kW3, b3, Rm, Sm):
    c0 = lambda i: (0, 0)
    return pl.pallas_call(
        _tc_edge1_body,
        grid=(_GRID,),
        in_specs=[
            pl.BlockSpec((_RB, 512), lambda i: (i, 0)),
            pl.BlockSpec((_RB, 128), lambda i: (i, 0)),
            pl.BlockSpec((64, 256), c0),
            pl.BlockSpec((1, 256), c0),
            pl.BlockSpec((16, 256), c0),
            pl.BlockSpec((256, 16), c0),
        ],
        out_specs=pl.BlockSpec((_RB, 128), lambda i: (i, 0)),
        out_shape=jax.ShapeDtypeStruct(((_E * _L) // 128, 128), jnp.float32),
    )(h2, xj, kW3, b3, Rm, Sm)


_NR = (_N * _L) // 128


def _tc_root_body(x_ref, a_ref, b_ref, da_ref, db_ref, rk_ref, bias_ref, out_ref):
    deg = jnp.maximum(da_ref[...] + db_ref[...], 1.0)
    agg = (a_ref[...] + b_ref[...]) / deg
    xw = jnp.dot(x_ref[...], rk_ref[...], preferred_element_type=jnp.float32)
    out_ref[...] = jax.nn.gelu(xw + agg + bias_ref[...])


def _tc_root(x, agg_a, agg_b, deg_a, deg_b, rootk, bias128):
    full = lambda: (0, 0)
    return pl.pallas_call(
        _tc_root_body,
        grid=(),
        in_specs=[
            pl.BlockSpec((_NR, 128), full),
            pl.BlockSpec((_NR, 128), full),
            pl.BlockSpec((_NR, 128), full),
            pl.BlockSpec((_NR, 128), full),
            pl.BlockSpec((_NR, 128), full),
            pl.BlockSpec((128, 128), full),
            pl.BlockSpec((1, 128), full),
        ],
        out_specs=pl.BlockSpec((_NR, 128), full),
        out_shape=jax.ShapeDtypeStruct((_NR, 128), jnp.float32),
    )(x, agg_a, agg_b, deg_a, deg_b, rootk, bias128)


def kernel(nodes, edge_index, edge_attr, kW1, kb1, kW2, kb2, kW3, kb3,
           root0, bias0, root1, bias1):
    src2d = edge_index[0].astype(jnp.int32).reshape(_E // _CH, _CH)
    dst2d = edge_index[1].astype(jnp.int32).reshape(_E // _CH, _CH)
    ea128 = edge_attr.reshape((_E * _L) // 128, 128)

    bf = jnp.bfloat16
    Rm = jnp.asarray((np.arange(256)[None, :] // 16 == np.arange(16)[:, None])
                     .astype(np.float32), dtype=bf)
    Sm = jnp.asarray((np.arange(256)[:, None] % 16 == np.arange(16)[None, :])
                     .astype(np.float32), dtype=bf)
    z = jnp.zeros((_NP, _L), jnp.float32)
    ones = jnp.ones((_CH, _L), jnp.float32)
    b1 = kb1.astype(bf).reshape(1, 64)
    b2 = kb2.astype(bf).reshape(1, 64)
    b3 = kb3.astype(bf).reshape(1, 256)
    eye8 = jnp.eye(8, dtype=jnp.float32)

    x128 = nodes.reshape(_NR, 128)
    degp = None
    h2 = None
    for root, bias, first in ((root0, bias0, True), (root1, bias1, False)):
        xj = _get_sc_gather()(x128.reshape(_N, _L), src2d)
        xj128 = xj.reshape((_E * _L) // 128, 128)
        if first:
            msg, h2 = _tc_edge0(ea128, xj128, kW1.astype(bf), b1,
                                kW2.astype(bf), b2, kW3.astype(bf), b3, Rm, Sm)
        else:
            msg = _tc_edge1(h2, xj128, kW3.astype(bf), b3, Rm, Sm)
        if first:
            aggp, degp = _make_sc_scatter(True)(msg.reshape(_E, _L), dst2d, z, ones)
        else:
            aggp = _make_sc_scatter(False)(msg.reshape(_E, _L), dst2d, z, ones)
        aggp = aggp.reshape(2, (_NP * _L) // 128, 128)
        degr = degp.reshape(2, (_NP * _L) // 128, 128)
        rootk = jnp.kron(eye8, root)
        bias128 = jnp.tile(bias, 8).reshape(1, 128)
        x128 = _tc_root(x128, aggp[0, :_NR], aggp[1, :_NR],
                        degr[0, :_NR], degr[1, :_NR], rootk, bias128)
    return x128.reshape(_N, _L)
